# Initial kernel scaffold; baseline (speedup 1.0000x reference)
import jax
import jax.numpy as jnp
from jax.experimental import pallas as pl


def kernel(y, emb, eW1, eb1, eW2, eb2, eW3, eb3, dW1, db1, dW2, db2, dW3, db3):
    raise NotImplementedError("write your pallas kernel here")


# speedup vs baseline: 1.6164x; 1.6164x over previous
import jax
import jax.numpy as jnp
from jax.experimental import pallas as pl
from jax.experimental.pallas import tpu as pltpu

_CODE_SEQ = 30
_MOTION_DIM = 84
_LATENT = 1024
_C = 128
_D = 16
_IN_DIM = _CODE_SEQ * _MOTION_DIM
_CB = _C * _D

_PREC = None
_HI = jax.lax.Precision.HIGHEST

_DN = (((1,), (1,)), ((), ()))


def _elu(x):
    return jnp.where(x > 0, x, jnp.exp(x) - 1.0)


def _mlp_body(x_ref, w1_ref, b1_ref, w2_ref, b2_ref, w3_ref, b3_ref, o_ref):
    h = jax.lax.dot_general(x_ref[...], w1_ref[...], _DN, precision=_PREC,
                            preferred_element_type=jnp.float32) + b1_ref[...]
    h = _elu(h)
    h = jax.lax.dot_general(h, w2_ref[...], _DN, precision=_PREC,
                            preferred_element_type=jnp.float32) + b2_ref[...]
    h = _elu(h)
    o_ref[...] = jax.lax.dot_general(h, w3_ref[...], _DN, precision=_PREC,
                                     preferred_element_type=jnp.float32) + b3_ref[...]


def _vq_body(zeg_ref, emb_ref, zq_ref):
    zeg = zeg_ref[...]
    emb = emb_ref[...]
    cross = jax.lax.dot_general(zeg, emb, _DN, precision=_PREC,
                                preferred_element_type=jnp.float32)
    e2 = jax.lax.dot_general(jnp.ones((1, _D), jnp.float32), emb * emb, _DN,
                             precision=_HI,
                             preferred_element_type=jnp.float32)
    s = jnp.sum(zeg * zeg, axis=1, keepdims=True)
    dist = (s - 2.0 * cross) + e2
    m = jnp.min(dist, axis=1, keepdims=True)
    ci = jax.lax.broadcasted_iota(jnp.int32, dist.shape, 1)
    idx = jnp.min(jnp.where(dist == m, ci, _C), axis=1, keepdims=True)
    oh = (ci == idx).astype(jnp.float32)
    zq_ref[...] = jax.lax.dot_general(oh, emb, (((1,), (0,)), ((), ())),
                                      precision=_HI,
                                      preferred_element_type=jnp.float32)


def _full(shape):
    return pl.BlockSpec(shape, lambda i: (0,) * len(shape))


def _mlp(x, w1, b1, w2, b2, w3, b3, rows_per_step):
    n, out_d = x.shape[0], w3.shape[0]
    grid = (n // rows_per_step,)
    return pl.pallas_call(
        _mlp_body,
        grid=grid,
        in_specs=[
            pl.BlockSpec((rows_per_step, x.shape[1]), lambda i: (i, 0)),
            _full(w1.shape), _full((1, w1.shape[0])),
            _full(w2.shape), _full((1, w2.shape[0])),
            _full(w3.shape), _full((1, w3.shape[0])),
        ],
        out_specs=pl.BlockSpec((rows_per_step, out_d), lambda i: (i, 0)),
        out_shape=jax.ShapeDtypeStruct((n, out_d), jnp.float32),
        compiler_params=pltpu.CompilerParams(
            dimension_semantics=("parallel",)),
    )(x, w1, b1.reshape(1, -1), w2, b2.reshape(1, -1), w3, b3.reshape(1, -1))


def _vq(zeg, emb, rows_per_step):
    n = zeg.shape[0]
    return pl.pallas_call(
        _vq_body,
        grid=(n // rows_per_step,),
        in_specs=[
            pl.BlockSpec((rows_per_step, _D), lambda i: (i, 0)),
            _full(emb.shape),
        ],
        out_specs=pl.BlockSpec((rows_per_step, _D), lambda i: (i, 0)),
        out_shape=jax.ShapeDtypeStruct((n, _D), jnp.float32),
        compiler_params=pltpu.CompilerParams(
            dimension_semantics=("parallel",)),
    )(zeg, emb)


def kernel(y, emb, eW1, eb1, eW2, eb2, eW3, eb3, dW1, db1, dW2, db2, dW3, db3):
    batch, seq_len = y.shape[0], y.shape[1]
    padding = _CODE_SEQ - seq_len % _CODE_SEQ
    if padding < _CODE_SEQ:
        pad = jnp.zeros((batch, padding, y.shape[-1]), dtype=jnp.float32)
        y = jnp.concatenate([y, pad], axis=1)
    x = y.reshape(-1, _IN_DIM)
    n = x.shape[0]
    n_m = n // batch

    r_enc = 400 if n % 400 == 0 else n
    ze = _mlp(x, eW1, eb1, eW2, eb2, eW3, eb3, r_enc)

    zeg = ze.reshape(n * _C, _D)
    r_vq = 8192 if (n * _C) % 8192 == 0 else n * _C
    zq_flat = _vq(zeg, emb, r_vq)
    zq = zq_flat.reshape(n, _CB)

    out = _mlp(zq, dW1, db1, dW2, db2, dW3, db3, r_enc)

    out = out.reshape(batch, -1, _MOTION_DIM)[:, :seq_len, :]
    return (out,
            ze.reshape(batch, n_m, _CB),
            zq.reshape(batch, n_m, _CB))


# speedup vs baseline: 2.9506x; 1.8253x over previous
import jax
import jax.numpy as jnp
from jax.experimental import pallas as pl
from jax.experimental.pallas import tpu as pltpu

_CODE_SEQ = 30
_MOTION_DIM = 84
_LATENT = 1024
_C = 128
_D = 16
_IN_DIM = _CODE_SEQ * _MOTION_DIM
_CB = _C * _D

_PREC = None
_HI = jax.lax.Precision.HIGHEST

_DN = (((1,), (1,)), ((), ()))


def _elu(x):
    return jnp.where(x > 0, x, jnp.exp(x) - 1.0)


def _mlp_body(x_ref, w1_ref, b1_ref, w2_ref, b2_ref, w3_ref, b3_ref, o_ref):
    h = jax.lax.dot_general(x_ref[...], w1_ref[...], _DN, precision=_PREC,
                            preferred_element_type=jnp.float32) + b1_ref[...]
    h = _elu(h)
    h = jax.lax.dot_general(h, w2_ref[...], _DN, precision=_PREC,
                            preferred_element_type=jnp.float32) + b2_ref[...]
    h = _elu(h)
    o_ref[...] = jax.lax.dot_general(h, w3_ref[...], _DN, precision=_PREC,
                                     preferred_element_type=jnp.float32) + b3_ref[...]


def _vq_body(ze_ref, embt_ref, zq_ref):
    rows = ze_ref.shape[0]
    grp = _CB // _C
    nslot = _C // grp
    mc = rows * (_CB // _C)
    zem = ze_ref[...].reshape(mc, _C)
    embt = embt_ref[...]
    l_i = jax.lax.broadcasted_iota(jnp.int32, (_C, nslot * _C), 0)
    jc_i = jax.lax.broadcasted_iota(jnp.int32, (_C, nslot * _C), 1)
    mask = (l_i // grp) == (jc_i // _C)
    t8 = jnp.concatenate([embt] * nslot, axis=0)
    t88 = jnp.concatenate([t8] * nslot, axis=1)
    bmat = jnp.where(mask, t88, 0.0)
    seg = jnp.where(mask, 1.0, 0.0)
    cross = jax.lax.dot_general(bmat, zem, (((0,), (1,)), ((), ())),
                                precision=_PREC,
                                preferred_element_type=jnp.float32)
    s = jax.lax.dot_general(seg, zem * zem, (((0,), (1,)), ((), ())),
                            precision=_PREC,
                            preferred_element_type=jnp.float32)
    e2 = jax.lax.dot_general(bmat * bmat, jnp.ones((1, _C), jnp.float32),
                             (((0,), (1,)), ((), ())),
                             precision=_HI,
                             preferred_element_type=jnp.float32)
    dist = ((s - 2.0 * cross) + e2).reshape(nslot, _C, mc)
    m = jnp.min(dist, axis=1, keepdims=True)
    ci = jax.lax.broadcasted_iota(jnp.int32, dist.shape, 1)
    idx = jnp.min(jnp.where(dist == m, ci, _C), axis=1, keepdims=True)
    oh = (ci == idx).astype(jnp.float32).reshape(nslot * _C, mc)
    zqm = jax.lax.dot_general(oh, bmat, (((0,), (1,)), ((), ())),
                              precision=_PREC,
                              preferred_element_type=jnp.float32)
    zq_ref[...] = zqm.reshape(rows, _CB)


def _full(shape):
    return pl.BlockSpec(shape, lambda i: (0,) * len(shape))


def _mlp(x, w1, b1, w2, b2, w3, b3, rows_per_step):
    n, out_d = x.shape[0], w3.shape[0]
    grid = (n // rows_per_step,)
    return pl.pallas_call(
        _mlp_body,
        grid=grid,
        in_specs=[
            pl.BlockSpec((rows_per_step, x.shape[1]), lambda i: (i, 0)),
            _full(w1.shape), _full((1, w1.shape[0])),
            _full(w2.shape), _full((1, w2.shape[0])),
            _full(w3.shape), _full((1, w3.shape[0])),
        ],
        out_specs=pl.BlockSpec((rows_per_step, out_d), lambda i: (i, 0)),
        out_shape=jax.ShapeDtypeStruct((n, out_d), jnp.float32),
        compiler_params=pltpu.CompilerParams(
            dimension_semantics=("parallel",)),
    )(x, w1, b1.reshape(1, -1), w2, b2.reshape(1, -1), w3, b3.reshape(1, -1))


def _vq(ze, emb, rows_per_step):
    n = ze.shape[0]
    embt = emb.T
    return pl.pallas_call(
        _vq_body,
        grid=(n // rows_per_step,),
        in_specs=[
            pl.BlockSpec((rows_per_step, _CB), lambda i: (i, 0)),
            _full(embt.shape),
        ],
        out_specs=pl.BlockSpec((rows_per_step, _CB), lambda i: (i, 0)),
        out_shape=jax.ShapeDtypeStruct((n, _CB), jnp.float32),
        compiler_params=pltpu.CompilerParams(
            dimension_semantics=("parallel",)),
    )(ze, embt)


def kernel(y, emb, eW1, eb1, eW2, eb2, eW3, eb3, dW1, db1, dW2, db2, dW3, db3):
    batch, seq_len = y.shape[0], y.shape[1]
    padding = _CODE_SEQ - seq_len % _CODE_SEQ
    if padding < _CODE_SEQ:
        pad = jnp.zeros((batch, padding, y.shape[-1]), dtype=jnp.float32)
        y = jnp.concatenate([y, pad], axis=1)
    x = y.reshape(-1, _IN_DIM)
    n = x.shape[0]
    n_m = n // batch

    r_enc = 400 if n % 400 == 0 else n
    ze = _mlp(x, eW1, eb1, eW2, eb2, eW3, eb3, r_enc)

    r_vq = 80 if n % 80 == 0 else n
    zq = _vq(ze, emb, r_vq)

    out = _mlp(zq, dW1, db1, dW2, db2, dW3, db3, r_enc)

    out = out.reshape(batch, -1, _MOTION_DIM)[:, :seq_len, :]
    return (out,
            ze.reshape(batch, n_m, _CB),
            zq.reshape(batch, n_m, _CB))


# speedup vs baseline: 3.1107x; 1.0543x over previous
import jax
import jax.numpy as jnp
from jax.experimental import pallas as pl
from jax.experimental.pallas import tpu as pltpu

_CODE_SEQ = 30
_MOTION_DIM = 84
_LATENT = 1024
_C = 128
_D = 16
_IN_DIM = _CODE_SEQ * _MOTION_DIM
_CB = _C * _D

_PREC = None
_HI = jax.lax.Precision.HIGHEST

_DN = (((1,), (1,)), ((), ()))


def _elu(x):
    return jnp.where(x > 0, x, jnp.exp(x) - 1.0)


def _mlp_body(x_ref, w1_ref, b1_ref, w2_ref, b2_ref, w3_ref, b3_ref, o_ref):
    x = x_ref[...]
    if x.ndim == 3:
        x = x.reshape(x.shape[0] * x.shape[1], x.shape[2])
    h = jax.lax.dot_general(x, w1_ref[...], _DN, precision=_PREC,
                            preferred_element_type=jnp.float32) + b1_ref[...]
    h = _elu(h)
    h = jax.lax.dot_general(h, w2_ref[...], _DN, precision=_PREC,
                            preferred_element_type=jnp.float32) + b2_ref[...]
    h = _elu(h)
    h = jax.lax.dot_general(h, w3_ref[...], _DN, precision=_PREC,
                            preferred_element_type=jnp.float32) + b3_ref[...]
    o_ref[...] = h.reshape(o_ref.shape)


def _vq_body(ze_ref, embt_ref, zq_ref):
    blk = ze_ref.shape
    grp = _CB // _C
    nslot = _C // grp
    mc = (blk[0] * blk[1] * blk[2]) // _C
    zem = ze_ref[...].reshape(mc, _C)
    embt = embt_ref[...]
    l_i = jax.lax.broadcasted_iota(jnp.int32, (_C, nslot * _C), 0)
    jc_i = jax.lax.broadcasted_iota(jnp.int32, (_C, nslot * _C), 1)
    mask = (l_i // grp) == (jc_i // _C)
    t8 = jnp.concatenate([embt] * nslot, axis=0)
    t88 = jnp.concatenate([t8] * nslot, axis=1)
    bmat = jnp.where(mask, t88, 0.0)
    seg = jnp.where(mask, 1.0, 0.0)
    cross = jax.lax.dot_general(bmat, zem, (((0,), (1,)), ((), ())),
                                precision=_PREC,
                                preferred_element_type=jnp.float32)
    s = jax.lax.dot_general(seg, zem * zem, (((0,), (1,)), ((), ())),
                            precision=_PREC,
                            preferred_element_type=jnp.float32)
    e2 = jax.lax.dot_general(bmat * bmat, jnp.ones((1, _C), jnp.float32),
                             (((0,), (1,)), ((), ())),
                             precision=_HI,
                             preferred_element_type=jnp.float32)
    dist = ((s - 2.0 * cross) + e2).reshape(nslot, _C, mc)
    m = jnp.min(dist, axis=1, keepdims=True)
    ci = jax.lax.broadcasted_iota(jnp.int32, dist.shape, 1)
    idx = jnp.min(jnp.where(dist == m, ci, _C), axis=1, keepdims=True)
    oh = (ci == idx).astype(jnp.float32).reshape(nslot * _C, mc)
    zqm = jax.lax.dot_general(oh, bmat, (((0,), (1,)), ((), ())),
                              precision=_PREC,
                              preferred_element_type=jnp.float32)
    zq_ref[...] = zqm.reshape(zq_ref.shape)


def _full(shape):
    return pl.BlockSpec(shape, lambda i: (0,) * len(shape))


def _row_spec(block_shape):
    nd = len(block_shape)
    return pl.BlockSpec(block_shape, lambda i: (i,) + (0,) * (nd - 1))


def _mlp(x, w1, b1, w2, b2, w3, b3, in_block, out_block, out_shape):
    grid = (x.shape[0] // in_block[0],)
    return pl.pallas_call(
        _mlp_body,
        grid=grid,
        in_specs=[
            _row_spec(in_block),
            _full(w1.shape), _full((1, w1.shape[0])),
            _full(w2.shape), _full((1, w2.shape[0])),
            _full(w3.shape), _full((1, w3.shape[0])),
        ],
        out_specs=_row_spec(out_block),
        out_shape=jax.ShapeDtypeStruct(out_shape, jnp.float32),
        compiler_params=pltpu.CompilerParams(
            dimension_semantics=("parallel",)),
    )(x, w1, b1.reshape(1, -1), w2, b2.reshape(1, -1), w3, b3.reshape(1, -1))


def _vq(ze3, emb):
    batch, n_m = ze3.shape[0], ze3.shape[1]
    vb = 2 if batch % 2 == 0 else 1
    embt = emb.T
    return pl.pallas_call(
        _vq_body,
        grid=(batch // vb,),
        in_specs=[
            _row_spec((vb, n_m, _CB)),
            _full(embt.shape),
        ],
        out_specs=_row_spec((vb, n_m, _CB)),
        out_shape=jax.ShapeDtypeStruct((batch, n_m, _CB), jnp.float32),
        compiler_params=pltpu.CompilerParams(
            dimension_semantics=("parallel",)),
    )(ze3, embt)


def kernel(y, emb, eW1, eb1, eW2, eb2, eW3, eb3, dW1, db1, dW2, db2, dW3, db3):
    batch, seq_len = y.shape[0], y.shape[1]
    padding = _CODE_SEQ - seq_len % _CODE_SEQ
    if padding < _CODE_SEQ:
        pad = jnp.zeros((batch, padding, y.shape[-1]), dtype=jnp.float32)
        y = jnp.concatenate([y, pad], axis=1)
    x = y.reshape(-1, _IN_DIM)
    n = x.shape[0]
    n_m = n // batch
    bb = 8 if batch % 8 == 0 else 1
    rows = bb * n_m

    ze3 = _mlp(x, eW1, eb1, eW2, eb2, eW3, eb3,
               (rows, _IN_DIM), (bb, n_m, _CB), (batch, n_m, _CB))

    zq3 = _vq(ze3, emb)

    out = _mlp(zq3, dW1, db1, dW2, db2, dW3, db3,
               (bb, n_m, _CB), (rows, _CODE_SEQ, _MOTION_DIM),
               (n, _CODE_SEQ, _MOTION_DIM))
    out = out.reshape(batch, -1, _MOTION_DIM)[:, :seq_len, :]
    return (out, ze3, zq3)
